# baseline (speedup 1.0000x reference)
import functools

import jax
import jax.numpy as jnp
from jax.experimental import pallas as pl
from jax.experimental.pallas import tpu as pltpu

_N = 2
_CBLK = 16 // _N


@functools.partial(
    pl.pallas_call,
    in_specs=[
        pl.BlockSpec(memory_space=pltpu.SMEM),
        pl.BlockSpec(memory_space=pl.ANY),
    ],
    out_specs=pl.BlockSpec(memory_space=pl.ANY),
    scratch_shapes=[
        pltpu.VMEM((_N, _CBLK, 64, 64), jnp.float32),
        pltpu.SemaphoreType.DMA,
        pltpu.SemaphoreType.DMA,
    ],
    out_shape=jax.ShapeDtypeStruct((1, 16, 64, 64), jnp.float32),
)
def _row_copy(idx_ref, x_hbm, o_hbm, buf, sem_in, sem_out):
    i = idx_ref[0]
    cin = [
        pltpu.make_async_copy(
            x_hbm.at[i, pl.ds(k * _CBLK, _CBLK)], buf.at[k], sem_in
        )
        for k in range(_N)
    ]
    cout = [
        pltpu.make_async_copy(
            buf.at[k], o_hbm.at[0, pl.ds(k * _CBLK, _CBLK)], sem_out
        )
        for k in range(_N)
    ]
    for c in cin:
        c.start()
    for k in range(_N):
        cin[k].wait()
        cout[k].start()
    for c in cout:
        c.wait()


def kernel(noises, i):
    idx = jnp.asarray(i, jnp.int32).reshape(1)
    return _row_copy(idx, noises)


# speedup vs baseline: 1.0200x; 1.0200x over previous
import functools

import jax
import jax.numpy as jnp
from jax.experimental import pallas as pl
from jax.experimental.pallas import tpu as pltpu

_N = 4
_CBLK = 16 // _N


@functools.partial(
    pl.pallas_call,
    in_specs=[
        pl.BlockSpec(memory_space=pltpu.SMEM),
        pl.BlockSpec(memory_space=pl.ANY),
    ],
    out_specs=pl.BlockSpec(memory_space=pl.ANY),
    scratch_shapes=[
        pltpu.VMEM((_N, _CBLK, 64, 64), jnp.float32),
        pltpu.SemaphoreType.DMA,
        pltpu.SemaphoreType.DMA,
    ],
    out_shape=jax.ShapeDtypeStruct((1, 16, 64, 64), jnp.float32),
)
def _row_copy(idx_ref, x_hbm, o_hbm, buf, sem_in, sem_out):
    i = idx_ref[0]
    cin = [
        pltpu.make_async_copy(
            x_hbm.at[i, pl.ds(k * _CBLK, _CBLK)], buf.at[k], sem_in
        )
        for k in range(_N)
    ]
    cout = [
        pltpu.make_async_copy(
            buf.at[k], o_hbm.at[0, pl.ds(k * _CBLK, _CBLK)], sem_out
        )
        for k in range(_N)
    ]
    for c in cin:
        c.start()
    for k in range(_N):
        cin[k].wait()
        cout[k].start()
    for c in cout:
        c.wait()


def kernel(noises, i):
    idx = jnp.asarray(i, jnp.int32).reshape(1)
    return _row_copy(idx, noises)
